# Initial kernel scaffold; baseline (speedup 1.0000x reference)
import functools

import jax
import jax.numpy as jnp
from jax.experimental import pallas as pl
from jax.experimental.pallas import tpu as pltpu

_MERGE_THRESHOLD = 0.85
_MIN_STRENGTH = 0.001
_STRENGTH_DECAY = 0.999
_WRITE_ALPHA = 0.25
_WRITE_BETA = 0.25
_BIG = jnp.int32(1 << 30)


def _body(wk_ref, wv_ref, ws_ref, k2_ref, v2_ref, age_ref, st_ref,
          ko_ref, vo_ref, ageo_ref, sto_ref, slot_ref, sim_ref):
    bb = k2_ref.shape[0]
    k2 = k2_ref.shape[1]
    lanes = k2_ref.shape[2]
    d = lanes // 2

    wk2 = wk_ref[...]
    li = jax.lax.broadcasted_iota(jnp.int32, (bb, lanes), 1)
    lo2 = li < d

    wk_sq = jnp.where(lo2, wk2 * wk2, 0.0)
    wk_nrm = jnp.sqrt(jnp.sum(wk_sq, axis=1, keepdims=True)) + 1e-6
    wkn = wk2 / wk_nrm

    kb = k2_ref[...]
    li3 = jax.lax.broadcasted_iota(jnp.int32, (bb, k2, lanes), 2)
    lo3 = li3 < d

    prod = kb * wkn[:, None, :]
    ksq = kb * kb
    d0 = jnp.sum(jnp.where(lo3, prod, 0.0), axis=2)
    d1 = jnp.sum(jnp.where(lo3, 0.0, prod), axis=2)
    s0 = jnp.sum(jnp.where(lo3, ksq, 0.0), axis=2)
    s1 = jnp.sum(jnp.where(lo3, 0.0, ksq), axis=2)
    sim0 = d0 / (jnp.sqrt(s0) + 1e-6)
    sim1 = d1 / (jnp.sqrt(s1) + 1e-6)

    best = jnp.maximum(jnp.max(sim0, axis=1, keepdims=True),
                       jnp.max(sim1, axis=1, keepdims=True))
    ji = jax.lax.broadcasted_iota(jnp.int32, (bb, k2), 1)
    ie = jnp.min(jnp.where(sim0 == best, 2 * ji, _BIG), axis=1, keepdims=True)
    io = jnp.min(jnp.where(sim1 == best, 2 * ji + 1, _BIG), axis=1, keepdims=True)
    best_idx = jnp.minimum(ie, io)

    age = age_ref[...]
    st = st_ref[...]
    ascore = age + (1.0 - jnp.clip(st, 0.0, 1.0)) * 0.01
    amax = jnp.max(ascore, axis=1, keepdims=True)
    ki = jax.lax.broadcasted_iota(jnp.int32, age.shape, 1)
    lru = jnp.min(jnp.where(ascore == amax, ki, _BIG), axis=1, keepdims=True)

    slot = jnp.where(best > _MERGE_THRESHOLD, best_idx, lru)
    at_slot = ki == slot

    ws = jnp.clip(ws_ref[...], 0.0, 1.0)
    ageo_ref[...] = jnp.where(at_slot, 0.0, age + 1.0)
    sdec = st * _STRENGTH_DECAY
    prev = jnp.sum(jnp.where(at_slot, sdec, 0.0), axis=1, keepdims=True)
    upd = jnp.clip(prev + ws * (1.0 - prev), _MIN_STRENGTH, 1.0)
    sto_ref[...] = jnp.where(at_slot, upd, sdec)

    jsel = slot // 2
    hsel = slot % 2
    j3 = jax.lax.broadcasted_iota(jnp.int32, (bb, k2, lanes), 1)
    h3 = (li3 >= d).astype(jnp.int32)
    sel = (j3 == jsel[:, :, None]) & (h3 == hsel[:, :, None])

    alpha = _WRITE_ALPHA * ws
    new_k = (1.0 - alpha)[:, :, None] * kb + alpha[:, :, None] * wk2[:, None, :]
    nk_sq = jnp.sum(jnp.where(sel, new_k * new_k, 0.0), axis=2)
    nk_nrm = jnp.sqrt(jnp.sum(nk_sq, axis=1, keepdims=True)) + 1e-6
    ko_ref[...] = jnp.where(sel, new_k / nk_nrm[:, :, None], kb)

    vb = v2_ref[...]
    wv2 = wv_ref[...]
    beta = _WRITE_BETA * ws
    new_v = (1.0 - beta)[:, :, None] * vb + beta[:, :, None] * wv2[:, None, :]
    vo_ref[...] = jnp.where(sel, new_v, vb)

    slot_ref[...] = slot
    sim_ref[...] = best


@functools.partial(jax.jit, static_argnames=("bb", "interpret"))
def _run(write_key, write_val, write_strength, epi_keys, epi_vals, epi_age,
         epi_strength, bb=64, interpret=False):
    b, k, d = epi_keys.shape
    lanes = 2 * d
    k2v = k // 2
    wk2 = jnp.concatenate([write_key, write_key], axis=1)
    wv2 = jnp.concatenate([write_val, write_val], axis=1)
    keys2 = epi_keys.reshape(b, k2v, lanes)
    vals2 = epi_vals.reshape(b, k2v, lanes)

    grid = (b // bb,)
    row2 = pl.BlockSpec((bb, lanes), lambda i: (i, 0))
    rowk = pl.BlockSpec((bb, k), lambda i: (i, 0))
    row1 = pl.BlockSpec((bb, 1), lambda i: (i, 0))
    big = pl.BlockSpec((bb, k2v, lanes), lambda i: (i, 0, 0))

    outs = pl.pallas_call(
        _body,
        grid=grid,
        in_specs=[row2, row2, row1, big, big, rowk, rowk],
        out_specs=[big, big, rowk, rowk,
                   pl.BlockSpec((bb, 1), lambda i: (i, 0)),
                   pl.BlockSpec((bb, 1), lambda i: (i, 0))],
        out_shape=[
            jax.ShapeDtypeStruct((b, k2v, lanes), jnp.float32),
            jax.ShapeDtypeStruct((b, k2v, lanes), jnp.float32),
            jax.ShapeDtypeStruct((b, k), jnp.float32),
            jax.ShapeDtypeStruct((b, k), jnp.float32),
            jax.ShapeDtypeStruct((b, 1), jnp.int32),
            jax.ShapeDtypeStruct((b, 1), jnp.float32),
        ],
        compiler_params=pltpu.CompilerParams(
            dimension_semantics=("arbitrary",)),
        interpret=interpret,
    )(wk2, wv2, write_strength, keys2, vals2, epi_age, epi_strength)

    ko, vo, ageo, sto, slot, sim = outs
    return (ko.reshape(b, k, d), vo.reshape(b, k, d), ageo, sto,
            slot.reshape(b), sim.reshape(b))


def kernel(write_key, write_val, write_strength, epi_keys, epi_vals,
           epi_age, epi_strength):
    return _run(write_key, write_val, write_strength, epi_keys, epi_vals,
                epi_age, epi_strength)


# speedup vs baseline: 1.0479x; 1.0479x over previous
import functools

import jax
import jax.numpy as jnp
from jax.experimental import pallas as pl
from jax.experimental.pallas import tpu as pltpu

_MERGE_THRESHOLD = 0.85
_MIN_STRENGTH = 0.001
_STRENGTH_DECAY = 0.999
_WRITE_ALPHA = 0.25
_WRITE_BETA = 0.25
_BIG = 1 << 30


def _body(wk_ref, wv_ref, ws_ref, k2_ref, v2_ref, age_ref, st_ref,
          ko_ref, vo_ref, ageo_ref, sto_ref, slot_ref, sim_ref):
    bb = k2_ref.shape[0]
    k2 = k2_ref.shape[1]
    lanes = k2_ref.shape[2]
    d = lanes // 2

    wk2 = wk_ref[...]
    li = jax.lax.broadcasted_iota(jnp.int32, (bb, lanes), 1)
    lo2 = li < d

    wk_sq = jnp.where(lo2, wk2 * wk2, 0.0)
    wk_nrm = jnp.sqrt(jnp.sum(wk_sq, axis=1, keepdims=True)) + 1e-6
    wkn = wk2 / wk_nrm

    kb = k2_ref[...]
    li3 = jax.lax.broadcasted_iota(jnp.int32, (bb, k2, lanes), 2)
    lo3 = li3 < d

    prod = kb * wkn[:, None, :]
    ksq = kb * kb
    d0 = jnp.sum(jnp.where(lo3, prod, 0.0), axis=2)
    d1 = jnp.sum(jnp.where(lo3, 0.0, prod), axis=2)
    s0 = jnp.sum(jnp.where(lo3, ksq, 0.0), axis=2)
    s1 = jnp.sum(jnp.where(lo3, 0.0, ksq), axis=2)
    sim0 = d0 / (jnp.sqrt(s0) + 1e-6)
    sim1 = d1 / (jnp.sqrt(s1) + 1e-6)

    best = jnp.maximum(jnp.max(sim0, axis=1, keepdims=True),
                       jnp.max(sim1, axis=1, keepdims=True))
    ji = jax.lax.broadcasted_iota(jnp.int32, (bb, k2), 1)
    ie = jnp.min(jnp.where(sim0 == best, 2 * ji, _BIG), axis=1, keepdims=True)
    io = jnp.min(jnp.where(sim1 == best, 2 * ji + 1, _BIG), axis=1, keepdims=True)
    best_idx = jnp.minimum(ie, io)

    age = age_ref[...]
    st = st_ref[...]
    ascore = age + (1.0 - jnp.clip(st, 0.0, 1.0)) * 0.01
    amax = jnp.max(ascore, axis=1, keepdims=True)
    ki = jax.lax.broadcasted_iota(jnp.int32, age.shape, 1)
    lru = jnp.min(jnp.where(ascore == amax, ki, _BIG), axis=1, keepdims=True)

    slot = jnp.where(best > _MERGE_THRESHOLD, best_idx, lru)
    at_slot = ki == slot

    ws = jnp.clip(ws_ref[...], 0.0, 1.0)
    ageo_ref[...] = jnp.where(at_slot, 0.0, age + 1.0)
    sdec = st * _STRENGTH_DECAY
    prev = jnp.sum(jnp.where(at_slot, sdec, 0.0), axis=1, keepdims=True)
    upd = jnp.clip(prev + ws * (1.0 - prev), _MIN_STRENGTH, 1.0)
    sto_ref[...] = jnp.where(at_slot, upd, sdec)

    jsel = slot // 2
    hsel = slot % 2
    j3 = jax.lax.broadcasted_iota(jnp.int32, (bb, k2, lanes), 1)
    h3 = (li3 >= d).astype(jnp.int32)
    sel = (j3 == jsel[:, :, None]) & (h3 == hsel[:, :, None])

    alpha = _WRITE_ALPHA * ws
    new_k = (1.0 - alpha)[:, :, None] * kb + alpha[:, :, None] * wk2[:, None, :]
    nk_sq = jnp.sum(jnp.where(sel, new_k * new_k, 0.0), axis=2)
    nk_nrm = jnp.sqrt(jnp.sum(nk_sq, axis=1, keepdims=True)) + 1e-6
    ko_ref[...] = jnp.where(sel, new_k / nk_nrm[:, :, None], kb)

    vb = v2_ref[...]
    wv2 = wv_ref[...]
    beta = _WRITE_BETA * ws
    new_v = (1.0 - beta)[:, :, None] * vb + beta[:, :, None] * wv2[:, None, :]
    vo_ref[...] = jnp.where(sel, new_v, vb)

    slot_ref[...] = slot
    sim_ref[...] = best


@functools.partial(jax.jit, static_argnames=("bb", "interpret"))
def _run(write_key, write_val, write_strength, epi_keys, epi_vals, epi_age,
         epi_strength, bb=64, interpret=False):
    b, k, d = epi_keys.shape
    lanes = 2 * d
    k2v = k // 2
    wk2 = jnp.concatenate([write_key, write_key], axis=1)
    wv2 = jnp.concatenate([write_val, write_val], axis=1)
    keys2 = epi_keys.reshape(b, k2v, lanes)
    vals2 = epi_vals.reshape(b, k2v, lanes)

    grid = (b // bb,)
    row2 = pl.BlockSpec((bb, lanes), lambda i: (i, 0))
    rowk = pl.BlockSpec((bb, k), lambda i: (i, 0))
    row1 = pl.BlockSpec((bb, 1), lambda i: (i, 0))
    big = pl.BlockSpec((bb, k2v, lanes), lambda i: (i, 0, 0))

    outs = pl.pallas_call(
        _body,
        grid=grid,
        in_specs=[row2, row2, row1, big, big, rowk, rowk],
        out_specs=[big, big, rowk, rowk,
                   pl.BlockSpec((bb, 1), lambda i: (i, 0)),
                   pl.BlockSpec((bb, 1), lambda i: (i, 0))],
        out_shape=[
            jax.ShapeDtypeStruct((b, k2v, lanes), jnp.float32),
            jax.ShapeDtypeStruct((b, k2v, lanes), jnp.float32),
            jax.ShapeDtypeStruct((b, k), jnp.float32),
            jax.ShapeDtypeStruct((b, k), jnp.float32),
            jax.ShapeDtypeStruct((b, 1), jnp.int32),
            jax.ShapeDtypeStruct((b, 1), jnp.float32),
        ],
        compiler_params=pltpu.CompilerParams(
            dimension_semantics=("arbitrary",)),
        interpret=interpret,
    )(wk2, wv2, write_strength, keys2, vals2, epi_age, epi_strength)

    ko, vo, ageo, sto, slot, sim = outs
    return (ko.reshape(b, k, d), vo.reshape(b, k, d), ageo, sto,
            slot.reshape(b), sim.reshape(b))


def kernel(write_key, write_val, write_strength, epi_keys, epi_vals,
           epi_age, epi_strength):
    return _run(write_key, write_val, write_strength, epi_keys, epi_vals,
                epi_age, epi_strength)


# speedup vs baseline: 1.1009x; 1.0505x over previous
import functools

import jax
import jax.numpy as jnp
from jax.experimental import pallas as pl
from jax.experimental.pallas import tpu as pltpu

_MERGE_THRESHOLD = 0.85
_MIN_STRENGTH = 0.001
_STRENGTH_DECAY = 0.999
_WRITE_ALPHA = 0.25
_WRITE_BETA = 0.25
_BIG = 1 << 30


def _body(wk_ref, wv_ref, ws_ref, kb_ref, vb_ref, age_ref, st_ref,
          ko_ref, vo_ref, ageo_ref, sto_ref, slot_ref, sim_ref):
    bb, k, d = kb_ref.shape

    wk = wk_ref[...]
    wk_nrm = jnp.sqrt(jnp.sum(wk * wk, axis=1, keepdims=True)) + 1e-6
    wkn = wk / wk_nrm

    kb = kb_ref[...]
    dots = jnp.sum(kb * wkn[:, None, :], axis=2)
    nsq = jnp.sum(kb * kb, axis=2)
    sim = dots / (jnp.sqrt(nsq) + 1e-6)

    best = jnp.max(sim, axis=1, keepdims=True)
    ki = jax.lax.broadcasted_iota(jnp.int32, (bb, k), 1)
    best_idx = jnp.min(jnp.where(sim == best, ki, _BIG), axis=1, keepdims=True)

    age = age_ref[...]
    st = st_ref[...]
    ascore = age + (1.0 - jnp.clip(st, 0.0, 1.0)) * 0.01
    amax = jnp.max(ascore, axis=1, keepdims=True)
    lru = jnp.min(jnp.where(ascore == amax, ki, _BIG), axis=1, keepdims=True)

    slot = jnp.where(best > _MERGE_THRESHOLD, best_idx, lru)
    at_slot = ki == slot

    ws = jnp.clip(ws_ref[...], 0.0, 1.0)
    ageo_ref[...] = jnp.where(at_slot, 0.0, age + 1.0)
    sdec = st * _STRENGTH_DECAY
    prev = jnp.sum(jnp.where(at_slot, sdec, 0.0), axis=1, keepdims=True)
    upd = jnp.clip(prev + ws * (1.0 - prev), _MIN_STRENGTH, 1.0)
    sto_ref[...] = jnp.where(at_slot, upd, sdec)

    k3 = jax.lax.broadcasted_iota(jnp.int32, (bb, k, d), 1)
    sel = k3 == slot[:, :, None]

    alpha = _WRITE_ALPHA * ws
    old_k = jnp.sum(jnp.where(sel, kb, 0.0), axis=1)
    new_k = (1.0 - alpha) * old_k + alpha * wk
    nk_nrm = jnp.sqrt(jnp.sum(new_k * new_k, axis=1, keepdims=True)) + 1e-6
    new_kn = new_k / nk_nrm
    ko_ref[...] = jnp.where(sel, new_kn[:, None, :], kb)

    vb = vb_ref[...]
    wv = wv_ref[...]
    beta = _WRITE_BETA * ws
    old_v = jnp.sum(jnp.where(sel, vb, 0.0), axis=1)
    new_v = (1.0 - beta) * old_v + beta * wv
    vo_ref[...] = jnp.where(sel, new_v[:, None, :], vb)

    slot_ref[...] = slot
    sim_ref[...] = best


@functools.partial(jax.jit, static_argnames=("bb", "interpret"))
def _run(write_key, write_val, write_strength, epi_keys, epi_vals, epi_age,
         epi_strength, bb=64, interpret=False):
    b, k, d = epi_keys.shape

    grid = (b // bb,)
    rowd = pl.BlockSpec((bb, d), lambda i: (i, 0))
    rowk = pl.BlockSpec((bb, k), lambda i: (i, 0))
    row1 = pl.BlockSpec((bb, 1), lambda i: (i, 0))
    big = pl.BlockSpec((bb, k, d), lambda i: (i, 0, 0))

    outs = pl.pallas_call(
        _body,
        grid=grid,
        in_specs=[rowd, rowd, row1, big, big, rowk, rowk],
        out_specs=[big, big, rowk, rowk, row1, row1],
        out_shape=[
            jax.ShapeDtypeStruct((b, k, d), jnp.float32),
            jax.ShapeDtypeStruct((b, k, d), jnp.float32),
            jax.ShapeDtypeStruct((b, k), jnp.float32),
            jax.ShapeDtypeStruct((b, k), jnp.float32),
            jax.ShapeDtypeStruct((b, 1), jnp.int32),
            jax.ShapeDtypeStruct((b, 1), jnp.float32),
        ],
        compiler_params=pltpu.CompilerParams(
            dimension_semantics=("arbitrary",)),
        interpret=interpret,
    )(write_key, write_val, write_strength, epi_keys, epi_vals, epi_age,
      epi_strength)

    ko, vo, ageo, sto, slot, sim = outs
    return (ko, vo, ageo, sto, slot.reshape(b), sim.reshape(b))


def kernel(write_key, write_val, write_strength, epi_keys, epi_vals,
           epi_age, epi_strength):
    return _run(write_key, write_val, write_strength, epi_keys, epi_vals,
                epi_age, epi_strength)


# speedup vs baseline: 5.9009x; 5.3602x over previous
import functools

import jax
import jax.numpy as jnp
from jax.experimental import pallas as pl
from jax.experimental.pallas import tpu as pltpu

_MERGE_THRESHOLD = 0.85
_MIN_STRENGTH = 0.001
_STRENGTH_DECAY = 0.999
_WRITE_ALPHA = 0.25
_WRITE_BETA = 0.25
_BIG = 1 << 30


def _body(wk_ref, wv_ref, ws_ref, kb_ref, vb_ref, age_ref, st_ref,
          ko_ref, vo_ref, ageo_ref, sto_ref, slot_ref, sim_ref):
    bb, d, k = kb_ref.shape

    wk = wk_ref[...]
    wk_nrm = jnp.sqrt(jnp.sum(wk * wk, axis=1, keepdims=True)) + 1e-6
    wkn = wk / wk_nrm

    kb = kb_ref[...]
    prod = kb * wkn[:, :, None]
    dots = jnp.sum(prod, axis=1)
    nsq = jnp.sum(kb * kb, axis=1)
    sim = dots / (jnp.sqrt(nsq) + 1e-6)

    best = jnp.max(sim, axis=1, keepdims=True)
    ki = jax.lax.broadcasted_iota(jnp.int32, (bb, k), 1)
    best_idx = jnp.min(jnp.where(sim == best, ki, _BIG), axis=1, keepdims=True)

    age = age_ref[...]
    st = st_ref[...]
    ascore = age + (1.0 - jnp.clip(st, 0.0, 1.0)) * 0.01
    amax = jnp.max(ascore, axis=1, keepdims=True)
    lru = jnp.min(jnp.where(ascore == amax, ki, _BIG), axis=1, keepdims=True)

    slot = jnp.where(best > _MERGE_THRESHOLD, best_idx, lru)
    at_slot = ki == slot

    ws = jnp.clip(ws_ref[...], 0.0, 1.0)
    ageo_ref[...] = jnp.where(at_slot, 0.0, age + 1.0)
    sdec = st * _STRENGTH_DECAY
    prev = jnp.sum(jnp.where(at_slot, sdec, 0.0), axis=1, keepdims=True)
    upd = jnp.clip(prev + ws * (1.0 - prev), _MIN_STRENGTH, 1.0)
    sto_ref[...] = jnp.where(at_slot, upd, sdec)

    sel = at_slot[:, None, :]

    alpha = _WRITE_ALPHA * ws
    old_k = jnp.sum(jnp.where(sel, kb, 0.0), axis=2)
    new_k = (1.0 - alpha) * old_k + alpha * wk
    nk_nrm = jnp.sqrt(jnp.sum(new_k * new_k, axis=1, keepdims=True)) + 1e-6
    new_kn = new_k / nk_nrm
    ko_ref[...] = jnp.where(sel, new_kn[:, :, None], kb)

    vb = vb_ref[...]
    wv = wv_ref[...]
    beta = _WRITE_BETA * ws
    old_v = jnp.sum(jnp.where(sel, vb, 0.0), axis=2)
    new_v = (1.0 - beta) * old_v + beta * wv
    vo_ref[...] = jnp.where(sel, new_v[:, :, None], vb)

    slot_ref[...] = slot
    sim_ref[...] = best


@functools.partial(jax.jit, static_argnames=("bb", "interpret"))
def _run(write_key, write_val, write_strength, epi_keys, epi_vals, epi_age,
         epi_strength, bb=128, interpret=False):
    b, k, d = epi_keys.shape
    ekt = epi_keys.transpose(0, 2, 1)
    evt = epi_vals.transpose(0, 2, 1)

    grid = (b // bb,)
    rowd = pl.BlockSpec((bb, d), lambda i: (i, 0))
    rowk = pl.BlockSpec((bb, k), lambda i: (i, 0))
    row1 = pl.BlockSpec((bb, 1), lambda i: (i, 0))
    big = pl.BlockSpec((bb, d, k), lambda i: (i, 0, 0))

    outs = pl.pallas_call(
        _body,
        grid=grid,
        in_specs=[rowd, rowd, row1, big, big, rowk, rowk],
        out_specs=[big, big, rowk, rowk, row1, row1],
        out_shape=[
            jax.ShapeDtypeStruct((b, d, k), jnp.float32),
            jax.ShapeDtypeStruct((b, d, k), jnp.float32),
            jax.ShapeDtypeStruct((b, k), jnp.float32),
            jax.ShapeDtypeStruct((b, k), jnp.float32),
            jax.ShapeDtypeStruct((b, 1), jnp.int32),
            jax.ShapeDtypeStruct((b, 1), jnp.float32),
        ],
        compiler_params=pltpu.CompilerParams(
            dimension_semantics=("arbitrary",)),
        interpret=interpret,
    )(write_key, write_val, write_strength, ekt, evt, epi_age, epi_strength)

    ko, vo, ageo, sto, slot, sim = outs
    return (ko.transpose(0, 2, 1), vo.transpose(0, 2, 1), ageo, sto,
            slot.reshape(b), sim.reshape(b))


def kernel(write_key, write_val, write_strength, epi_keys, epi_vals,
           epi_age, epi_strength):
    return _run(write_key, write_val, write_strength, epi_keys, epi_vals,
                epi_age, epi_strength)


# speedup vs baseline: 6.5693x; 1.1133x over previous
import functools

import jax
import jax.numpy as jnp
from jax.experimental import pallas as pl
from jax.experimental.pallas import tpu as pltpu

_MERGE_THRESHOLD = 0.85
_MIN_STRENGTH = 0.001
_STRENGTH_DECAY = 0.999
_WRITE_ALPHA = 0.25
_WRITE_BETA = 0.25
_BIG = 1 << 30


def _body(wk_ref, wv_ref, ws_ref, kb_ref, vb_ref, age_ref, st_ref,
          ko_ref, vo_ref, ageo_ref, sto_ref, slot_ref, sim_ref):
    bb, d, k = kb_ref.shape

    wk = wk_ref[...]
    wksq = jnp.sum(wk * wk, axis=1, keepdims=True)
    wk_nrm = jnp.sqrt(wksq) + 1e-6
    wkn3 = (wk / wk_nrm)[:, :, None]

    kb = kb_ref[...]
    dots = jnp.sum(kb * wkn3, axis=1)
    nsq = jnp.sum(kb * kb, axis=1)
    sim = dots / (jnp.sqrt(nsq) + 1e-6)

    best = jnp.max(sim, axis=1, keepdims=True)
    ki = jax.lax.broadcasted_iota(jnp.int32, (bb, k), 1)
    best_idx = jnp.min(jnp.where(sim == best, ki, _BIG), axis=1, keepdims=True)

    age = age_ref[...]
    st = st_ref[...]
    ascore = age + (1.0 - jnp.clip(st, 0.0, 1.0)) * 0.01
    amax = jnp.max(ascore, axis=1, keepdims=True)
    lru = jnp.min(jnp.where(ascore == amax, ki, _BIG), axis=1, keepdims=True)

    slot = jnp.where(best > _MERGE_THRESHOLD, best_idx, lru)
    at_slot = ki == slot

    ws = jnp.clip(ws_ref[...], 0.0, 1.0)
    ageo_ref[...] = jnp.where(at_slot, 0.0, age + 1.0)
    sdec = st * _STRENGTH_DECAY
    prev = jnp.sum(jnp.where(at_slot, sdec, 0.0), axis=1, keepdims=True)
    upd = jnp.clip(prev + ws * (1.0 - prev), _MIN_STRENGTH, 1.0)
    sto_ref[...] = jnp.where(at_slot, upd, sdec)

    sel = at_slot[:, None, :]

    alpha = _WRITE_ALPHA * ws
    oma = 1.0 - alpha
    dots_at = jnp.sum(jnp.where(at_slot, dots, 0.0), axis=1, keepdims=True)
    nsq_at = jnp.sum(jnp.where(at_slot, nsq, 0.0), axis=1, keepdims=True)
    dotw_at = dots_at * wk_nrm
    nk2 = oma * oma * nsq_at + 2.0 * alpha * oma * dotw_at + alpha * alpha * wksq
    rcp_k = 1.0 / (jnp.sqrt(nk2) + 1e-6)

    coef_k = (alpha * wk_nrm)[:, :, None]
    blend_k = (oma[:, :, None] * kb + coef_k * wkn3) * rcp_k[:, :, None]
    ko_ref[...] = jnp.where(sel, blend_k, kb)

    vb = vb_ref[...]
    wv3 = wv_ref[...][:, :, None]
    beta = _WRITE_BETA * ws
    blend_v = (1.0 - beta)[:, :, None] * vb + beta[:, :, None] * wv3
    vo_ref[...] = jnp.where(sel, blend_v, vb)

    slot_ref[...] = slot
    sim_ref[...] = best


@functools.partial(jax.jit, static_argnames=("bb", "interpret"))
def _run(write_key, write_val, write_strength, epi_keys, epi_vals, epi_age,
         epi_strength, bb=128, interpret=False):
    b, k, d = epi_keys.shape
    ekt = epi_keys.transpose(0, 2, 1)
    evt = epi_vals.transpose(0, 2, 1)

    grid = (b // bb,)
    rowd = pl.BlockSpec((bb, d), lambda i: (i, 0))
    rowk = pl.BlockSpec((bb, k), lambda i: (i, 0))
    row1 = pl.BlockSpec((bb, 1), lambda i: (i, 0))
    big = pl.BlockSpec((bb, d, k), lambda i: (i, 0, 0))

    outs = pl.pallas_call(
        _body,
        grid=grid,
        in_specs=[rowd, rowd, row1, big, big, rowk, rowk],
        out_specs=[big, big, rowk, rowk, row1, row1],
        out_shape=[
            jax.ShapeDtypeStruct((b, d, k), jnp.float32),
            jax.ShapeDtypeStruct((b, d, k), jnp.float32),
            jax.ShapeDtypeStruct((b, k), jnp.float32),
            jax.ShapeDtypeStruct((b, k), jnp.float32),
            jax.ShapeDtypeStruct((b, 1), jnp.int32),
            jax.ShapeDtypeStruct((b, 1), jnp.float32),
        ],
        compiler_params=pltpu.CompilerParams(
            dimension_semantics=("arbitrary",)),
        interpret=interpret,
    )(write_key, write_val, write_strength, ekt, evt, epi_age, epi_strength)

    ko, vo, ageo, sto, slot, sim = outs
    return (ko.transpose(0, 2, 1), vo.transpose(0, 2, 1), ageo, sto,
            slot.reshape(b), sim.reshape(b))


def kernel(write_key, write_val, write_strength, epi_keys, epi_vals,
           epi_age, epi_strength):
    return _run(write_key, write_val, write_strength, epi_keys, epi_vals,
                epi_age, epi_strength)
